# Initial kernel scaffold; baseline (speedup 1.0000x reference)
import jax
import jax.numpy as jnp
from jax.experimental import pallas as pl


def kernel(features, adj_indices, red_indices, blue_indices, W, b, lamda):
    raise NotImplementedError("write your pallas kernel here")


# speedup vs baseline: 12.1282x; 12.1282x over previous
import functools

import jax
import jax.numpy as jnp
from jax import lax
from jax.experimental import pallas as pl
from jax.experimental.pallas import tpu as pltpu
from jax.experimental.pallas import tpu_sc as plsc

_N = 10000
_D = 128
_K = 16
_ROWS = 2000

_NC = 2
_NS = 16
_NW = _NC * _NS
_C = 100
_CHUNKS = 200

_SELU_SCALE = 1.0507009873554805
_SELU_ALPHA = 1.6732632423543772


def _tc_body(f_ref, w_ref, b_ref, a_ref, cs_ref, pool_ref):
    i = pl.program_id(0)
    f = f_ref[...]
    logits = jnp.dot(f, w_ref[...], preferred_element_type=jnp.float32) + b_ref[...]
    mx = jnp.max(logits, axis=1, keepdims=True)
    e = jnp.exp(logits - mx)
    a = e / jnp.sum(e, axis=1, keepdims=True)
    a_ref[...] = a

    @pl.when(i == 0)
    def _init():
        cs_ref[...] = jnp.zeros_like(cs_ref)
        pool_ref[...] = jnp.zeros_like(pool_ref)

    cs_ref[...] += jnp.sum(a, axis=0, keepdims=True)
    pool_ref[...] += lax.dot_general(f, a, (((0,), (0,)), ((), ())),
                                     preferred_element_type=jnp.float32)

    @pl.when(i == pl.num_programs(0) - 1)
    def _finish():
        p = pool_ref[...] / cs_ref[...]
        pool_ref[...] = jnp.where(
            p > 0.0, _SELU_SCALE * p, _SELU_SCALE * _SELU_ALPHA * (jnp.exp(p) - 1.0))


_tc_call = pl.pallas_call(
    _tc_body,
    grid=(_N // _ROWS,),
    in_specs=[
        pl.BlockSpec((_ROWS, _D), lambda i: (i, 0)),
        pl.BlockSpec((_D, _K), lambda i: (0, 0)),
        pl.BlockSpec((1, _K), lambda i: (0, 0)),
    ],
    out_specs=[
        pl.BlockSpec((_ROWS, _K), lambda i: (i, 0)),
        pl.BlockSpec((1, _K), lambda i: (0, 0)),
        pl.BlockSpec((_D, _K), lambda i: (0, 0)),
    ],
    out_shape=[
        jax.ShapeDtypeStruct((_N, _K), jnp.float32),
        jax.ShapeDtypeStruct((1, _K), jnp.float32),
        jax.ShapeDtypeStruct((_D, _K), jnp.float32),
    ],
)


@functools.partial(
    pl.kernel,
    out_type=jax.ShapeDtypeStruct((_NW, 8, _K), jnp.float32),
    mesh=plsc.VectorSubcoreMesh(core_axis_name="c", subcore_axis_name="s"),
    compiler_params=pltpu.CompilerParams(use_tc_tiling_on_sc=False),
    scratch_types=[
        pltpu.VMEM((_CHUNKS, _C), jnp.int32),
        pltpu.VMEM((_CHUNKS, _C), jnp.int32),
        pltpu.VMEM((_C, _K), jnp.float32),
        pltpu.VMEM((_C, _K), jnp.float32),
        pltpu.VMEM((8, _K), jnp.float32),
        pltpu.SemaphoreType.DMA,
        pltpu.SemaphoreType.DMA,
    ],
)
def _sc_call(a_hbm, adj_s, adj_d, red_s, red_d, blue_s, blue_d, out_hbm,
             idx_s, idx_d, rows_s, rows_d, acc, sem_s, sem_d):
    wid = lax.axis_index("s") * _NC + lax.axis_index("c")

    def process(src_hbm, dst_hbm, widx):
        pltpu.sync_copy(src_hbm.at[widx], idx_s)
        pltpu.sync_copy(dst_hbm.at[widx], idx_d)
        acc[...] = jnp.zeros((8, _K), jnp.float32)

        def chunk(c, carry):
            cp_s = pltpu.async_copy(a_hbm.at[idx_s.at[c]], rows_s, sem_s)
            cp_d = pltpu.async_copy(a_hbm.at[idx_d.at[c]], rows_d, sem_d)
            cp_s.wait()
            cp_d.wait()
            t = [jnp.zeros((_K,), jnp.float32) for _ in range(4)]
            s = [jnp.zeros((_K,), jnp.float32) for _ in range(4)]
            for i in range(_C):
                rs = rows_s[i, :]
                rd = rows_d[i, :]
                t[i % 4] = t[i % 4] + rs * rd
                s[i % 4] = s[i % 4] + rd
            acc[0, :] += (t[0] + t[1]) + (t[2] + t[3])
            acc[1, :] += (s[0] + s[1]) + (s[2] + s[3])
            return carry

        lax.fori_loop(0, _CHUNKS, chunk, 0)
        pltpu.sync_copy(acc, out_hbm.at[wid])

    @pl.when(wid < 16)
    def _adj():
        process(adj_s, adj_d, wid)

    @pl.when(jnp.logical_and(wid >= 16, wid < 24))
    def _red():
        process(red_s, red_d, wid - 16)

    @pl.when(wid >= 24)
    def _blue():
        process(blue_s, blue_d, wid - 24)


def kernel(features, adj_indices, red_indices, blue_indices, W, b, lamda):
    n = features.shape[0]
    m = jnp.float32(adj_indices.shape[1])
    ne_half = jnp.float32(red_indices.shape[1])

    assignments, cs2, pooled_t = _tc_call(features, W, b.reshape(1, _K))

    partials = _sc_call(
        assignments,
        adj_indices[0].reshape(16, _CHUNKS, _C), adj_indices[1].reshape(16, _CHUNKS, _C),
        red_indices[0].reshape(8, _CHUNKS, _C), red_indices[1].reshape(8, _CHUNKS, _C),
        blue_indices[0].reshape(8, _CHUNKS, _C), blue_indices[1].reshape(8, _CHUNKS, _C),
    )

    dot_p = partials[:, 0, :]
    s_p = partials[:, 1, :]
    t_adj = jnp.sum(dot_p[:16])
    s_adj = jnp.sum(s_p[:16], axis=0)
    t_red = jnp.sum(dot_p[16:24])
    s_red = jnp.sum(s_p[16:24], axis=0)
    t_blue = jnp.sum(dot_p[24:32])
    s_blue = jnp.sum(s_p[24:32], axis=0)

    def term(t, s, ne):
        return -(t - jnp.dot(s, s) / (2.0 * ne)) / (2.0 * m)

    red_loss = term(t_red, s_red, ne_half)
    blue_loss = term(t_blue, s_blue, ne_half)
    spectral_loss = term(t_adj, s_adj, m)

    cs = cs2[0]
    collapse_loss = (jnp.sqrt(jnp.sum(cs * cs)) / n * jnp.sqrt(jnp.float32(_K)) - 1.0)
    fair_term = jnp.abs(lamda * (red_loss - blue_loss))
    total_loss = (jnp.where(lamda != 0, fair_term, jnp.float32(0.0))
                  + jnp.where(lamda != 1, spectral_loss, jnp.float32(0.0))
                  + jnp.float32(0.1) * collapse_loss)

    features_pooled = pooled_t.T
    return (features_pooled, assignments, total_loss)


# speedup vs baseline: 16.4377x; 1.3553x over previous
import functools

import jax
import jax.numpy as jnp
from jax import lax
from jax.experimental import pallas as pl
from jax.experimental.pallas import tpu as pltpu
from jax.experimental.pallas import tpu_sc as plsc

_N = 10000
_D = 128
_K = 16
_ROWS = 2000

_NC = 2
_NS = 16
_NW = _NC * _NS
_C = 100
_CHUNKS = 200

_SELU_SCALE = 1.0507009873554805
_SELU_ALPHA = 1.6732632423543772


def _tc_body(f_ref, w_ref, b_ref, a_ref, cs_ref, pool_ref):
    i = pl.program_id(0)
    f = f_ref[...]
    logits = jnp.dot(f, w_ref[...], preferred_element_type=jnp.float32) + b_ref[...]
    mx = jnp.max(logits, axis=1, keepdims=True)
    e = jnp.exp(logits - mx)
    a = e / jnp.sum(e, axis=1, keepdims=True)
    a_ref[...] = a

    @pl.when(i == 0)
    def _init():
        cs_ref[...] = jnp.zeros_like(cs_ref)
        pool_ref[...] = jnp.zeros_like(pool_ref)

    cs_ref[...] += jnp.sum(a, axis=0, keepdims=True)
    pool_ref[...] += lax.dot_general(f, a, (((0,), (0,)), ((), ())),
                                     preferred_element_type=jnp.float32)

    @pl.when(i == pl.num_programs(0) - 1)
    def _finish():
        p = pool_ref[...] / cs_ref[...]
        pool_ref[...] = jnp.where(
            p > 0.0, _SELU_SCALE * p, _SELU_SCALE * _SELU_ALPHA * (jnp.exp(p) - 1.0))


_tc_call = pl.pallas_call(
    _tc_body,
    grid=(_N // _ROWS,),
    in_specs=[
        pl.BlockSpec((_ROWS, _D), lambda i: (i, 0)),
        pl.BlockSpec((_D, _K), lambda i: (0, 0)),
        pl.BlockSpec((1, _K), lambda i: (0, 0)),
    ],
    out_specs=[
        pl.BlockSpec((_ROWS, _K), lambda i: (i, 0)),
        pl.BlockSpec((1, _K), lambda i: (0, 0)),
        pl.BlockSpec((_D, _K), lambda i: (0, 0)),
    ],
    out_shape=[
        jax.ShapeDtypeStruct((_N, _K), jnp.float32),
        jax.ShapeDtypeStruct((1, _K), jnp.float32),
        jax.ShapeDtypeStruct((_D, _K), jnp.float32),
    ],
)


@functools.partial(
    pl.kernel,
    out_type=jax.ShapeDtypeStruct((_NW, 8, _K), jnp.float32),
    mesh=plsc.VectorSubcoreMesh(core_axis_name="c", subcore_axis_name="s"),
    compiler_params=pltpu.CompilerParams(use_tc_tiling_on_sc=False),
    scratch_types=[
        pltpu.VMEM((_CHUNKS, _C), jnp.int32),
        pltpu.VMEM((_CHUNKS, _C), jnp.int32),
        pltpu.VMEM((_C, _K), jnp.float32),
        pltpu.VMEM((_C, _K), jnp.float32),
        pltpu.VMEM((_C, _K), jnp.float32),
        pltpu.VMEM((_C, _K), jnp.float32),
        pltpu.VMEM((8, _K), jnp.float32),
        pltpu.SemaphoreType.DMA,
        pltpu.SemaphoreType.DMA,
    ],
)
def _sc_call(a_hbm, adj_s, adj_d, red_s, red_d, blue_s, blue_d, out_hbm,
             idx_s, idx_d, rows_s0, rows_d0, rows_s1, rows_d1, acc,
             sem0, sem1):
    wid = lax.axis_index("s") * _NC + lax.axis_index("c")

    def process(src_hbm, dst_hbm, widx):
        pltpu.sync_copy(src_hbm.at[widx], idx_s)
        pltpu.sync_copy(dst_hbm.at[widx], idx_d)
        acc[...] = jnp.zeros((8, _K), jnp.float32)

        def fire(c, rows_s, rows_d, sem):
            pltpu.async_copy(a_hbm.at[idx_s.at[c]], rows_s, sem)
            pltpu.async_copy(a_hbm.at[idx_d.at[c]], rows_d, sem)

        def wait(rows_s, rows_d, sem):
            pltpu.make_async_copy(a_hbm.at[idx_s.at[0]], rows_s, sem).wait()
            pltpu.make_async_copy(a_hbm.at[idx_d.at[0]], rows_d, sem).wait()

        def compute(rows_s, rows_d):
            t = [jnp.zeros((_K,), jnp.float32) for _ in range(4)]
            s = [jnp.zeros((_K,), jnp.float32) for _ in range(4)]
            for i in range(_C):
                rs = rows_s[i, :]
                rd = rows_d[i, :]
                t[i % 4] = t[i % 4] + rs * rd
                s[i % 4] = s[i % 4] + rd
            acc[0, :] += (t[0] + t[1]) + (t[2] + t[3])
            acc[1, :] += (s[0] + s[1]) + (s[2] + s[3])

        fire(0, rows_s0, rows_d0, sem0)

        def pair(j, carry):
            fire(2 * j + 1, rows_s1, rows_d1, sem1)
            wait(rows_s0, rows_d0, sem0)
            compute(rows_s0, rows_d0)

            @pl.when(j < _CHUNKS // 2 - 1)
            def _next():
                fire(2 * j + 2, rows_s0, rows_d0, sem0)

            wait(rows_s1, rows_d1, sem1)
            compute(rows_s1, rows_d1)
            return carry

        lax.fori_loop(0, _CHUNKS // 2, pair, 0)
        pltpu.sync_copy(acc, out_hbm.at[wid])

    @pl.when(wid < 16)
    def _adj():
        process(adj_s, adj_d, wid)

    @pl.when(jnp.logical_and(wid >= 16, wid < 24))
    def _red():
        process(red_s, red_d, wid - 16)

    @pl.when(wid >= 24)
    def _blue():
        process(blue_s, blue_d, wid - 24)


def kernel(features, adj_indices, red_indices, blue_indices, W, b, lamda):
    n = features.shape[0]
    m = jnp.float32(adj_indices.shape[1])
    ne_half = jnp.float32(red_indices.shape[1])

    assignments, cs2, pooled_t = _tc_call(features, W, b.reshape(1, _K))

    partials = _sc_call(
        assignments,
        adj_indices[0].reshape(16, _CHUNKS, _C), adj_indices[1].reshape(16, _CHUNKS, _C),
        red_indices[0].reshape(8, _CHUNKS, _C), red_indices[1].reshape(8, _CHUNKS, _C),
        blue_indices[0].reshape(8, _CHUNKS, _C), blue_indices[1].reshape(8, _CHUNKS, _C),
    )

    dot_p = partials[:, 0, :]
    s_p = partials[:, 1, :]
    t_adj = jnp.sum(dot_p[:16])
    s_adj = jnp.sum(s_p[:16], axis=0)
    t_red = jnp.sum(dot_p[16:24])
    s_red = jnp.sum(s_p[16:24], axis=0)
    t_blue = jnp.sum(dot_p[24:32])
    s_blue = jnp.sum(s_p[24:32], axis=0)

    def term(t, s, ne):
        return -(t - jnp.dot(s, s) / (2.0 * ne)) / (2.0 * m)

    red_loss = term(t_red, s_red, ne_half)
    blue_loss = term(t_blue, s_blue, ne_half)
    spectral_loss = term(t_adj, s_adj, m)

    cs = cs2[0]
    collapse_loss = (jnp.sqrt(jnp.sum(cs * cs)) / n * jnp.sqrt(jnp.float32(_K)) - 1.0)
    fair_term = jnp.abs(lamda * (red_loss - blue_loss))
    total_loss = (jnp.where(lamda != 0, fair_term, jnp.float32(0.0))
                  + jnp.where(lamda != 1, spectral_loss, jnp.float32(0.0))
                  + jnp.float32(0.1) * collapse_loss)

    features_pooled = pooled_t.T
    return (features_pooled, assignments, total_loss)


# speedup vs baseline: 21.7275x; 1.3218x over previous
import functools

import jax
import jax.numpy as jnp
from jax import lax
from jax.experimental import pallas as pl
from jax.experimental.pallas import tpu as pltpu
from jax.experimental.pallas import tpu_sc as plsc

_N = 10000
_D = 128
_K = 16

_NC = 2
_NS = 16
_NW = _NC * _NS
_C = 100
_CHUNKS = 200
_NBUF = 4

_SELU_SCALE = 1.0507009873554805
_SELU_ALPHA = 1.6732632423543772


def _tc_body(f_ref, w_ref, b_ref, pool_ref, a_ref, cs_ref):
    f = f_ref[...]
    logits = jnp.dot(f, w_ref[...], preferred_element_type=jnp.float32) + b_ref[...]
    mx = jnp.max(logits, axis=1, keepdims=True)
    e = jnp.exp(logits - mx)
    a = e / jnp.sum(e, axis=1, keepdims=True)
    a_ref[...] = a
    cs = jnp.sum(a, axis=0, keepdims=True)
    cs_ref[...] = cs
    ap = a / cs
    p = lax.dot_general(ap, f, (((0,), (0,)), ((), ())),
                        preferred_element_type=jnp.float32)
    pool_ref[...] = jnp.where(
        p > 0.0, _SELU_SCALE * p, _SELU_SCALE * _SELU_ALPHA * (jnp.exp(p) - 1.0))


_tc_call = pl.pallas_call(
    _tc_body,
    out_shape=[
        jax.ShapeDtypeStruct((_K, _D), jnp.float32),
        jax.ShapeDtypeStruct((_N, _K), jnp.float32),
        jax.ShapeDtypeStruct((1, _K), jnp.float32),
    ],
)


@functools.partial(
    pl.kernel,
    out_type=jax.ShapeDtypeStruct((_NW, 8, _K), jnp.float32),
    mesh=plsc.VectorSubcoreMesh(core_axis_name="c", subcore_axis_name="s"),
    compiler_params=pltpu.CompilerParams(use_tc_tiling_on_sc=False),
    scratch_types=[
        pltpu.VMEM((_CHUNKS, _C), jnp.int32),
        pltpu.VMEM((_CHUNKS, _C), jnp.int32),
        [pltpu.VMEM((_C, _K), jnp.float32) for _ in range(_NBUF)],
        [pltpu.VMEM((_C, _K), jnp.float32) for _ in range(_NBUF)],
        pltpu.VMEM((8, _K), jnp.float32),
        [pltpu.SemaphoreType.DMA for _ in range(_NBUF)],
    ],
)
def _sc_call(a_hbm, adj_s, adj_d, red_s, red_d, blue_s, blue_d, out_hbm,
             idx_s, idx_d, rows_s, rows_d, acc, sems):
    wid = lax.axis_index("s") * _NC + lax.axis_index("c")

    def process(src_hbm, dst_hbm, widx):
        pltpu.sync_copy(src_hbm.at[widx], idx_s)
        pltpu.sync_copy(dst_hbm.at[widx], idx_d)
        acc[...] = jnp.zeros((8, _K), jnp.float32)

        def fire(c, b):
            pltpu.async_copy(a_hbm.at[idx_s.at[c]], rows_s[b], sems[b])
            pltpu.async_copy(a_hbm.at[idx_d.at[c]], rows_d[b], sems[b])

        def wait(b):
            pltpu.make_async_copy(a_hbm.at[idx_s.at[0]], rows_s[b], sems[b]).wait()
            pltpu.make_async_copy(a_hbm.at[idx_d.at[0]], rows_d[b], sems[b]).wait()

        def compute(b):
            t = [jnp.zeros((_K,), jnp.float32) for _ in range(4)]
            s = [jnp.zeros((_K,), jnp.float32) for _ in range(4)]
            for i in range(_C):
                rs = rows_s[b][i, :]
                rd = rows_d[b][i, :]
                t[i % 4] = t[i % 4] + rs * rd
                s[i % 4] = s[i % 4] + rd
            acc[0, :] += (t[0] + t[1]) + (t[2] + t[3])
            acc[1, :] += (s[0] + s[1]) + (s[2] + s[3])

        for b in range(_NBUF):
            fire(b, b)

        def group(j, carry):
            for b in range(_NBUF):
                wait(b)
                compute(b)

                @pl.when(j < _CHUNKS // _NBUF - 1)
                def _next():
                    fire(_NBUF * (j + 1) + b, b)

            return carry

        lax.fori_loop(0, _CHUNKS // _NBUF, group, 0)
        pltpu.sync_copy(acc, out_hbm.at[wid])

    @pl.when(wid < 16)
    def _adj():
        process(adj_s, adj_d, wid)

    @pl.when(jnp.logical_and(wid >= 16, wid < 24))
    def _red():
        process(red_s, red_d, wid - 16)

    @pl.when(wid >= 24)
    def _blue():
        process(blue_s, blue_d, wid - 24)


def kernel(features, adj_indices, red_indices, blue_indices, W, b, lamda):
    n = features.shape[0]
    m = jnp.float32(adj_indices.shape[1])
    ne_half = jnp.float32(red_indices.shape[1])

    features_pooled, assignments, cs2 = _tc_call(features, W, b.reshape(1, _K))

    partials = _sc_call(
        assignments,
        adj_indices[0].reshape(16, _CHUNKS, _C), adj_indices[1].reshape(16, _CHUNKS, _C),
        red_indices[0].reshape(8, _CHUNKS, _C), red_indices[1].reshape(8, _CHUNKS, _C),
        blue_indices[0].reshape(8, _CHUNKS, _C), blue_indices[1].reshape(8, _CHUNKS, _C),
    )

    ps = jnp.sum(partials[:, 0:2, :].reshape(4, 8, 2, _K), axis=1)
    t_adj = jnp.sum(ps[0, 0] + ps[1, 0])
    s_adj = ps[0, 1] + ps[1, 1]
    t_red = jnp.sum(ps[2, 0])
    s_red = ps[2, 1]
    t_blue = jnp.sum(ps[3, 0])
    s_blue = ps[3, 1]

    def term(t, s, ne):
        return -(t - jnp.dot(s, s) / (2.0 * ne)) / (2.0 * m)

    red_loss = term(t_red, s_red, ne_half)
    blue_loss = term(t_blue, s_blue, ne_half)
    spectral_loss = term(t_adj, s_adj, m)

    cs = cs2[0]
    collapse_loss = (jnp.sqrt(jnp.sum(cs * cs)) / n * jnp.sqrt(jnp.float32(_K)) - 1.0)
    fair_term = jnp.abs(lamda * (red_loss - blue_loss))
    total_loss = (jnp.where(lamda != 0, fair_term, jnp.float32(0.0))
                  + jnp.where(lamda != 1, spectral_loss, jnp.float32(0.0))
                  + jnp.float32(0.1) * collapse_loss)

    return (features_pooled, assignments, total_loss)


# speedup vs baseline: 23.9137x; 1.1006x over previous
import functools

import jax
import jax.numpy as jnp
from jax import lax
from jax.experimental import pallas as pl
from jax.experimental.pallas import tpu as pltpu
from jax.experimental.pallas import tpu_sc as plsc

_N = 10000
_D = 128
_K = 16

_NC = 2
_NS = 16
_NW = _NC * _NS
_C = 125
_CHUNKS = 160
_NBUF = 4

_SELU_SCALE = 1.0507009873554805
_SELU_ALPHA = 1.6732632423543772


def _tc_body(f_ref, w_ref, b_ref, pool_ref, a_ref, cs_ref):
    f = f_ref[...]
    logits = jnp.dot(f, w_ref[...], preferred_element_type=jnp.float32) + b_ref[...]
    mx = jnp.max(logits, axis=1, keepdims=True)
    e = jnp.exp(logits - mx)
    a = e / jnp.sum(e, axis=1, keepdims=True)
    a_ref[...] = a
    cs = jnp.sum(a, axis=0, keepdims=True)
    cs_ref[...] = cs
    ap = a / cs
    p = lax.dot_general(ap, f, (((0,), (0,)), ((), ())),
                        preferred_element_type=jnp.float32)
    pool_ref[...] = jnp.where(
        p > 0.0, _SELU_SCALE * p, _SELU_SCALE * _SELU_ALPHA * (jnp.exp(p) - 1.0))


_tc_call = pl.pallas_call(
    _tc_body,
    out_shape=[
        jax.ShapeDtypeStruct((_K, _D), jnp.float32),
        jax.ShapeDtypeStruct((_N, _K), jnp.float32),
        jax.ShapeDtypeStruct((1, _K), jnp.float32),
    ],
)


@functools.partial(
    pl.kernel,
    out_type=jax.ShapeDtypeStruct((_NW * 8, _K), jnp.float32),
    mesh=plsc.VectorSubcoreMesh(core_axis_name="c", subcore_axis_name="s"),
    compiler_params=pltpu.CompilerParams(use_tc_tiling_on_sc=False),
    scratch_types=[
        pltpu.VMEM((_CHUNKS, _C), jnp.int32),
        pltpu.VMEM((_CHUNKS, _C), jnp.int32),
        [pltpu.VMEM((_C, _K), jnp.float32) for _ in range(_NBUF)],
        [pltpu.VMEM((_C, _K), jnp.float32) for _ in range(_NBUF)],
        pltpu.VMEM((8, _K), jnp.float32),
        [pltpu.SemaphoreType.DMA for _ in range(_NBUF)],
    ],
)
def _sc_call(a_hbm, src_hbm, dst_hbm, out_hbm,
             idx_s, idx_d, rows_s, rows_d, acc, sems):
    wid = lax.axis_index("s") * _NC + lax.axis_index("c")

    pltpu.sync_copy(src_hbm.at[wid], idx_s)
    pltpu.sync_copy(dst_hbm.at[wid], idx_d)
    acc[...] = jnp.zeros((8, _K), jnp.float32)

    def fire(c, b):
        pltpu.async_copy(a_hbm.at[idx_s.at[c]], rows_s[b], sems[b])
        pltpu.async_copy(a_hbm.at[idx_d.at[c]], rows_d[b], sems[b])

    def wait(b):
        pltpu.make_async_copy(a_hbm.at[idx_s.at[0]], rows_s[b], sems[b]).wait()
        pltpu.make_async_copy(a_hbm.at[idx_d.at[0]], rows_d[b], sems[b]).wait()

    def compute(b):
        t = [jnp.zeros((_K,), jnp.float32) for _ in range(4)]
        s = [jnp.zeros((_K,), jnp.float32) for _ in range(4)]
        for i in range(_C):
            rs = rows_s[b][i, :]
            rd = rows_d[b][i, :]
            t[i % 4] = t[i % 4] + rs * rd
            s[i % 4] = s[i % 4] + rd
        acc[0, :] += (t[0] + t[1]) + (t[2] + t[3])
        acc[1, :] += (s[0] + s[1]) + (s[2] + s[3])

    for b in range(_NBUF):
        fire(b, b)

    def group(j, carry):
        for b in range(_NBUF):
            wait(b)
            compute(b)

            @pl.when(j < _CHUNKS // _NBUF - 1)
            def _next():
                fire(_NBUF * (j + 1) + b, b)

        return carry

    lax.fori_loop(0, _CHUNKS // _NBUF, group, 0)
    pltpu.sync_copy(acc, out_hbm.at[pl.ds(wid * 8, 8)])


def _loss_body(p_ref, cs_ref, lam_ref, out_ref):
    p = p_ref[...]
    r = lax.broadcasted_iota(jnp.int32, (_NW * 8, _K), 0)
    isdot = (r % 8) == 0
    iss = (r % 8) == 1
    zero = jnp.zeros_like(p)

    m = jnp.float32(_CHUNKS * _C * 16)
    ne_half = jnp.float32(_CHUNKS * _C * 8)

    def term(sel, ne):
        t = jnp.sum(jnp.where(isdot & sel, p, zero))
        s = jnp.sum(jnp.where(iss & sel, p, zero), axis=0, keepdims=True)
        return -(t - jnp.sum(s * s) / (2.0 * ne)) / (2.0 * m)

    adj_loss = term(r < 128, m)
    red_loss = term((r >= 128) & (r < 192), ne_half)
    blue_loss = term(r >= 192, ne_half)

    cs = cs_ref[...]
    collapse_loss = (jnp.sqrt(jnp.sum(cs * cs)) / _N * jnp.sqrt(jnp.float32(_K))
                     - 1.0)
    lam = lam_ref[0, 0]
    lam_f = lam.astype(jnp.float32)
    fair_term = jnp.abs(lam_f * (red_loss - blue_loss))
    total = (jnp.where(lam != 0, fair_term, jnp.float32(0.0))
             + jnp.where(lam != 1, adj_loss, jnp.float32(0.0))
             + jnp.float32(0.1) * collapse_loss)
    out_ref[...] = jnp.full((1, 1), 0.0, jnp.float32) + total


_loss_call = pl.pallas_call(
    _loss_body,
    in_specs=[
        pl.BlockSpec(memory_space=pltpu.VMEM),
        pl.BlockSpec(memory_space=pltpu.VMEM),
        pl.BlockSpec(memory_space=pltpu.SMEM),
    ],
    out_specs=pl.BlockSpec(memory_space=pltpu.VMEM),
    out_shape=jax.ShapeDtypeStruct((1, 1), jnp.float32),
)


def kernel(features, adj_indices, red_indices, blue_indices, W, b, lamda):
    features_pooled, assignments, cs2 = _tc_call(features, W, b.reshape(1, _K))

    src_all = jnp.concatenate(
        [adj_indices[0], red_indices[0], blue_indices[0]]).reshape(_NW, _CHUNKS, _C)
    dst_all = jnp.concatenate(
        [adj_indices[1], red_indices[1], blue_indices[1]]).reshape(_NW, _CHUNKS, _C)

    partials = _sc_call(assignments, src_all, dst_all)

    lam = jnp.asarray(lamda, jnp.int32).reshape(1, 1)
    total_loss = _loss_call(partials, cs2, lam)[0, 0]

    return (features_pooled, assignments, total_loss)
